# baseline (speedup 1.0000x reference)
import functools

import jax
import jax.numpy as jnp
from jax import lax
from jax.experimental import pallas as pl
from jax.experimental.pallas import tpu as pltpu
from jax.experimental.pallas import tpu_sc as plsc

_BATCH = 16384
_HIST = 200
_DIM = 64

_info = plsc.get_sparse_core_info()
_NC, _NS = _info.num_cores, _info.num_subcores
_NW = _NC * _NS
_PER_W = _BATCH // _NW
_ROWS = 4
_STEPS = _PER_W // _ROWS
_NPAIR = _STEPS // 2


@functools.partial(
    pl.kernel,
    mesh=plsc.VectorSubcoreMesh(core_axis_name="c", subcore_axis_name="s"),
    out_type=jax.ShapeDtypeStruct((_BATCH, _HIST, _DIM), jnp.float32),
    scratch_types=[
        pltpu.VMEM((_ROWS, _HIST), jnp.int32),
        pltpu.VMEM((_ROWS, _HIST), jnp.int32),
        pltpu.VMEM((_ROWS, _HIST, _DIM), jnp.float32),
        pltpu.VMEM((_ROWS, _HIST, _DIM), jnp.float32),
        pltpu.SemaphoreType.DMA,
        pltpu.SemaphoreType.DMA,
        pltpu.SemaphoreType.DMA,
        pltpu.SemaphoreType.DMA,
    ],
    compiler_params=pltpu.CompilerParams(use_tc_tiling_on_sc=False),
)
def _sc_gather(idx_hbm, tab_hbm, out_hbm, idx_v0, idx_v1, rows_v0, rows_v1,
               gsem0, gsem1, wsem0, wsem1):
    wid = lax.axis_index("s") * _NC + lax.axis_index("c")
    base = wid * _PER_W

    class gather:
        def __init__(self, idx_v, rows_v, gsem):
            self.copies = [
                pltpu.make_async_copy(tab_hbm.at[idx_v.at[j]], rows_v.at[j], gsem)
                for j in range(_ROWS)
            ]

        def start(self):
            for c in self.copies:
                c.start()

        def wait(self):
            for c in self.copies:
                c.wait()

    def writeout(rows_v, off, wsem):
        return pltpu.make_async_copy(rows_v, out_hbm.at[pl.ds(off, _ROWS)], wsem)

    pltpu.sync_copy(idx_hbm.at[pl.ds(base, _ROWS)], idx_v0)
    gather(idx_v0, rows_v0, gsem0).start()

    def pair(i, carry):
        off0 = base + (2 * i) * _ROWS
        off1 = off0 + _ROWS

        gather(idx_v0, rows_v0, gsem0).wait()
        writeout(rows_v0, off0, wsem0).start()

        @pl.when(i > 0)
        def _():
            writeout(rows_v1, off0 - _ROWS, wsem1).wait()

        pltpu.sync_copy(idx_hbm.at[pl.ds(off1, _ROWS)], idx_v1)
        gather(idx_v1, rows_v1, gsem1).start()

        gather(idx_v1, rows_v1, gsem1).wait()
        writeout(rows_v1, off1, wsem1).start()

        writeout(rows_v0, off0, wsem0).wait()

        @pl.when(i < _NPAIR - 1)
        def _():
            pltpu.sync_copy(idx_hbm.at[pl.ds(off1 + _ROWS, _ROWS)], idx_v0)
            gather(idx_v0, rows_v0, gsem0).start()

        return carry

    lax.fori_loop(0, _NPAIR, pair, jnp.int32(0))

    writeout(rows_v1, base + (_STEPS - 1) * _ROWS, wsem1).wait()


def kernel(a, embed_weight):
    return _sc_gather(a.astype(jnp.int32), embed_weight)


# speedup vs baseline: 1.3040x; 1.3040x over previous
import functools

import jax
import jax.numpy as jnp
from jax import lax
from jax.experimental import pallas as pl
from jax.experimental.pallas import tpu as pltpu
from jax.experimental.pallas import tpu_sc as plsc

_BATCH = 16384
_HIST = 200
_DIM = 64

_info = plsc.get_sparse_core_info()
_NC, _NS = _info.num_cores, _info.num_subcores
_NW = _NC * _NS
_PER_W = _BATCH // _NW
_CROWS = 2
_CIDX = _CROWS * _HIST
_NCHUNK = _PER_W // _CROWS
_NOUTER = _NCHUNK // 2


@functools.partial(
    pl.kernel,
    mesh=plsc.VectorSubcoreMesh(core_axis_name="c", subcore_axis_name="s"),
    out_type=jax.ShapeDtypeStruct((_BATCH * _HIST, 128), jnp.float32),
    scratch_types=[
        pltpu.VMEM((_CIDX,), jnp.int32),
        pltpu.VMEM((_CIDX,), jnp.int32),
        pltpu.VMEM((_CIDX, 128), jnp.float32),
        pltpu.VMEM((_CIDX, 128), jnp.float32),
        pltpu.SemaphoreType.DMA,
        pltpu.SemaphoreType.DMA,
        pltpu.SemaphoreType.DMA,
        pltpu.SemaphoreType.DMA,
        pltpu.SemaphoreType.DMA,
        pltpu.SemaphoreType.DMA,
    ],
)
def _sc_gather(idx_hbm, tab_hbm, out_hbm, idx_v0, idx_v1, rows_v0, rows_v1,
               isem0, isem1, gsem0, gsem1, wsem0, wsem1):
    wid = lax.axis_index("s") * _NC + lax.axis_index("c")
    rbase = wid * _PER_W
    fbase = rbase * _HIST
    idx_bufs = (idx_v0, idx_v1)
    isems = (isem0, isem1)
    rows_bufs = (rows_v0, rows_v1)
    gsems = (gsem0, gsem1)
    wsems = (wsem0, wsem1)

    def idx_load(k, p):
        return pltpu.make_async_copy(
            idx_hbm.at[pl.ds(fbase + k * _CIDX, _CIDX)], idx_bufs[p], isems[p])

    def gather(p):
        return pltpu.make_async_copy(
            tab_hbm.at[idx_bufs[p]], rows_bufs[p], gsems[p])

    def writeouts(p, r0):
        return [
            pltpu.make_async_copy(
                rows_bufs[p], out_hbm.at[pl.ds(r0 * _HIST, _CIDX)], wsems[p])
        ]

    pltpu.sync_copy(idx_hbm.at[pl.ds(fbase, _CIDX)], idx_v0)
    gather(0).start()

    def pair(i, carry):
        for p in range(2):
            k = 2 * i + p
            r0 = rbase + k * _CROWS

            def prefetch():
                idx_load(k + 1, 1 - p).start()

            if p == 0:
                prefetch()
            else:
                pl.when(i < _NOUTER - 1)(prefetch)

            gather(p).wait()
            for d in writeouts(p, r0):
                d.start()

            def wait_prev():
                for d in writeouts(1 - p, r0 - _CROWS):
                    d.wait()

            if p == 0:
                pl.when(i > 0)(wait_prev)
            else:
                wait_prev()

            def start_next():
                idx_load(k + 1, 1 - p).wait()
                gather(1 - p).start()

            if p == 0:
                start_next()
            else:
                pl.when(i < _NOUTER - 1)(start_next)

        return carry

    lax.fori_loop(0, _NOUTER, pair, jnp.int32(0))

    for d in writeouts(1, rbase + (_NCHUNK - 1) * _CROWS):
        d.wait()


def kernel(a, embed_weight):
    tab128 = jnp.pad(embed_weight, ((0, 0), (0, 128 - _DIM)))
    out = _sc_gather(a.reshape(-1).astype(jnp.int32), tab128)
    return out[:, :_DIM].reshape(_BATCH, _HIST, _DIM)
